# baseline (speedup 1.0000x reference)
import functools

import jax
import jax.numpy as jnp
from jax import lax
from jax.experimental import pallas as pl
from jax.experimental.pallas import tpu as pltpu
from jax.experimental.pallas import tpu_sc as plsc


_BS = 2048


def _add_kernel(x_ref, emb_ref, o_ref):
    o_ref[0] = x_ref[0] + emb_ref[...]


def _kernel_tc(x, emb_weight):
    B, S, D = x.shape
    emb = emb_weight[:S]
    grid = (S // _BS, B)
    return pl.pallas_call(
        _add_kernel,
        grid=grid,
        in_specs=[
            pl.BlockSpec((1, _BS, D), lambda i, j: (j, i, 0)),
            pl.BlockSpec((_BS, D), lambda i, j: (i, 0)),
        ],
        out_specs=pl.BlockSpec((1, _BS, D), lambda i, j: (j, i, 0)),
        out_shape=jax.ShapeDtypeStruct((B, S, D), x.dtype),
    )(x, emb)


_NC = 2
_NS = 16
_NW = _NC * _NS
_R = 32


def _kernel_sc(x, emb_weight):
    B, S, D = x.shape
    x2d = x.reshape(B * S, D)
    emb = emb_weight[:S]
    seq_per_w = S // _NW
    n_chunks = seq_per_w // _R
    mesh = plsc.VectorSubcoreMesh(core_axis_name="c", subcore_axis_name="s")

    @functools.partial(
        pl.kernel,
        out_type=jax.ShapeDtypeStruct((B * S, D), jnp.float32),
        mesh=mesh,
        scratch_types=[
            pltpu.VMEM((_R, D), jnp.float32),
            pltpu.VMEM((_R, D), jnp.float32),
        ],
    )
    def k(x_hbm, emb_hbm, out_hbm, xbuf, ebuf):
        wid = lax.axis_index("s") * _NC + lax.axis_index("c")
        seq0 = wid * seq_per_w

        def chunk_body(c, _):
            s0 = seq0 + c * _R
            pltpu.sync_copy(emb_hbm.at[pl.ds(s0, _R)], ebuf)

            def batch_body(b, _):
                g0 = b * S + s0
                pltpu.sync_copy(x_hbm.at[pl.ds(g0, _R)], xbuf)

                def row_body(i, _):
                    for j in range(D // 16):
                        sl = pl.ds(j * 16, 16)
                        xbuf[i, sl] = xbuf[i, sl] + ebuf[i, sl]
                    return 0

                lax.fori_loop(0, _R, row_body, 0)
                pltpu.sync_copy(xbuf, out_hbm.at[pl.ds(g0, _R)])
                return 0

            lax.fori_loop(0, B, batch_body, 0)
            return 0

        lax.fori_loop(0, n_chunks, chunk_body, 0)

    return k(x2d, emb).reshape(B, S, D)


def kernel(x, emb_weight):
    return _kernel_sc(x, emb_weight)


# speedup vs baseline: 2.0370x; 2.0370x over previous
import functools

import jax
import jax.numpy as jnp
from jax import lax
from jax.experimental import pallas as pl
from jax.experimental.pallas import tpu as pltpu
from jax.experimental.pallas import tpu_sc as plsc


_BS = 2048
_TC_NBUF = None


def _add_kernel(x_ref, emb_ref, o_ref):
    o_ref[0] = x_ref[0] + emb_ref[...]


def _kernel_tc(x, emb_weight):
    B, S, D = x.shape
    emb = emb_weight[:S]
    grid = (S // _BS, B)
    nbuf = _TC_NBUF
    return pl.pallas_call(
        _add_kernel,
        grid=grid,
        in_specs=[
            pl.BlockSpec((1, _BS, D), lambda i, j: (j, i, 0),
                         pl.Buffered(buffer_count=nbuf) if nbuf else None),
            pl.BlockSpec((_BS, D), lambda i, j: (i, 0)),
        ],
        out_specs=pl.BlockSpec((1, _BS, D), lambda i, j: (j, i, 0),
                               pl.Buffered(buffer_count=nbuf) if nbuf else None),
        out_shape=jax.ShapeDtypeStruct((B, S, D), x.dtype),
    )(x, emb)


_NC = 2
_NS = 16
_NW = _NC * _NS
_R = 32


def _kernel_sc(x, emb_weight):
    B, S, D = x.shape
    x2d = x.reshape(B * S, D)
    emb = emb_weight[:S]
    seq_per_w = S // _NW
    n_chunks = seq_per_w // _R
    mesh = plsc.VectorSubcoreMesh(core_axis_name="c", subcore_axis_name="s")

    @functools.partial(
        pl.kernel,
        out_type=jax.ShapeDtypeStruct((B * S, D), jnp.float32),
        mesh=mesh,
        scratch_types=[
            pltpu.VMEM((_R, D), jnp.float32),
            pltpu.VMEM((_R, D), jnp.float32),
        ],
    )
    def k(x_hbm, emb_hbm, out_hbm, xbuf, ebuf):
        wid = lax.axis_index("s") * _NC + lax.axis_index("c")
        seq0 = wid * seq_per_w

        def chunk_body(c, _):
            s0 = seq0 + c * _R
            pltpu.sync_copy(emb_hbm.at[pl.ds(s0, _R)], ebuf)

            def batch_body(b, _):
                g0 = b * S + s0
                pltpu.sync_copy(x_hbm.at[pl.ds(g0, _R)], xbuf)

                def row_body(i, _):
                    for j in range(D // 16):
                        sl = pl.ds(j * 16, 16)
                        xbuf[i, sl] = xbuf[i, sl] + ebuf[i, sl]
                    return 0

                lax.fori_loop(0, _R, row_body, 0)
                pltpu.sync_copy(xbuf, out_hbm.at[pl.ds(g0, _R)])
                return 0

            lax.fori_loop(0, B, batch_body, 0)
            return 0

        lax.fori_loop(0, n_chunks, chunk_body, 0)

    return k(x2d, emb).reshape(B, S, D)


_SC_SEQ = 2048
_R_TC = 1024


from jax._src.pallas.mosaic.core import TensorCoreMesh as _TensorCoreMesh


class _HbmTensorCoreMesh(_TensorCoreMesh):
    @property
    def default_memory_space(self):
        return pltpu.HBM


def _make_tc_mesh(axis_name):
    base = pltpu.create_tensorcore_mesh(axis_name)
    return _HbmTensorCoreMesh(base.devices, base.axis_names)


def _kernel_hybrid(x, emb_weight):
    B, S, D = x.shape
    x2d = x.reshape(B * S, D)
    emb = emb_weight[:S]
    C = _SC_SEQ
    n_tc_chunks = (S - C) // _R_TC
    c_blk = C // _R_TC
    s_blk = S // _R_TC

    seq_per_w = C // _NW
    n_sc_chunks = seq_per_w // _R

    tc_mesh = _make_tc_mesh("tc")
    sc_mesh = plsc.VectorSubcoreMesh(core_axis_name="c", subcore_axis_name="s")

    def body_tc(x_hbm, emb_hbm, out_hbm):
        def inner(x_blk, emb_blk, out_blk):
            out_blk[...] = x_blk[...] + emb_blk[...]

        pltpu.emit_pipeline(
            inner,
            grid=(n_tc_chunks, B),
            in_specs=[
                pl.BlockSpec((_R_TC, D), lambda i, j: (j * s_blk + c_blk + i, 0)),
                pl.BlockSpec((_R_TC, D), lambda i, j: (c_blk + i, 0)),
            ],
            out_specs=[
                pl.BlockSpec((_R_TC, D), lambda i, j: (j * s_blk + c_blk + i, 0)),
            ],
        )(x_hbm, emb_hbm, out_hbm)

    def body_sc(x_hbm, emb_hbm, out_hbm):
        @functools.partial(
            pl.run_scoped,
            xbuf=pltpu.VMEM((_R, D), jnp.float32),
            ebuf=pltpu.VMEM((_R, D), jnp.float32),
        )
        def _(xbuf, ebuf):
            wid = lax.axis_index("s") * _NC + lax.axis_index("c")
            seq0 = wid * seq_per_w

            def chunk_body(c, _):
                s0 = seq0 + c * _R
                pltpu.sync_copy(emb_hbm.at[pl.ds(s0, _R)], ebuf)

                def batch_body(b, _):
                    g0 = b * S + s0
                    pltpu.sync_copy(x_hbm.at[pl.ds(g0, _R)], xbuf)

                    def row_body(i, _):
                        for j in range(D // 16):
                            sl = pl.ds(j * 16, 16)
                            xbuf[i, sl] = xbuf[i, sl] + ebuf[i, sl]
                        return 0

                    lax.fori_loop(0, _R, row_body, 0)
                    pltpu.sync_copy(xbuf, out_hbm.at[pl.ds(g0, _R)])
                    return 0

                lax.fori_loop(0, B, batch_body, 0)
                return 0

            lax.fori_loop(0, n_sc_chunks, chunk_body, 0)

    k = pl.kernel(
        [body_sc, body_tc],
        out_type=jax.ShapeDtypeStruct((B * S, D), jnp.float32),
        mesh=[sc_mesh, tc_mesh],
    )
    return k(x2d, emb).reshape(B, S, D)


_RP = 4
_NSLOT = 4


def _kernel_sc_pipe(x, emb_weight):
    B, S, D = x.shape
    x2d = x.reshape(B * S, D)
    emb = emb_weight[:S]
    seq_per_w = S // _NW
    n_chunks = seq_per_w // _RP
    mesh = plsc.VectorSubcoreMesh(core_axis_name="c", subcore_axis_name="s")

    @functools.partial(
        pl.kernel,
        out_type=jax.ShapeDtypeStruct((B * S, D), jnp.float32),
        mesh=mesh,
        scratch_types=[
            pltpu.VMEM((_NSLOT, B, _RP, D), jnp.float32),
            pltpu.VMEM((_NSLOT, _RP, D), jnp.float32),
            pltpu.SemaphoreType.DMA((_NSLOT,)),
            pltpu.SemaphoreType.DMA((_NSLOT,)),
            pltpu.SemaphoreType.DMA((_NSLOT,)),
        ],
    )
    def k(x_hbm, emb_hbm, out_hbm, xb, eb, sem_in, sem_e, sem_out):
        wid = lax.axis_index("s") * _NC + lax.axis_index("c")
        seq0 = wid * seq_per_w

        def start_in(slot, c):
            s0 = seq0 + c * _RP
            for b in range(B):
                pltpu.make_async_copy(
                    x_hbm.at[pl.ds(b * S + s0, _RP)], xb.at[slot, b],
                    sem_in.at[slot]).start()
            pltpu.make_async_copy(
                emb_hbm.at[pl.ds(s0, _RP)], eb.at[slot], sem_e.at[slot]).start()

        def wait_in(slot, c):
            s0 = seq0 + c * _RP
            for b in range(B):
                pltpu.make_async_copy(
                    x_hbm.at[pl.ds(b * S + s0, _RP)], xb.at[slot, b],
                    sem_in.at[slot]).wait()
            pltpu.make_async_copy(
                emb_hbm.at[pl.ds(s0, _RP)], eb.at[slot], sem_e.at[slot]).wait()

        def start_out(slot, c):
            s0 = seq0 + c * _RP
            for b in range(B):
                pltpu.make_async_copy(
                    xb.at[slot, b], out_hbm.at[pl.ds(b * S + s0, _RP)],
                    sem_out.at[slot]).start()

        def wait_out(slot, c):
            s0 = seq0 + c * _RP
            for b in range(B):
                pltpu.make_async_copy(
                    xb.at[slot, b], out_hbm.at[pl.ds(b * S + s0, _RP)],
                    sem_out.at[slot]).wait()

        def compute(slot):
            def col_body(ct, _):
                sl = pl.ds(ct * 16, 16)
                for r in range(_RP):
                    e = eb[slot, r, sl]
                    for b in range(B):
                        xb[slot, b, r, sl] = xb[slot, b, r, sl] + e
                return 0

            lax.fori_loop(0, D // 16, col_body, 0)

        start_in(0, 0)
        start_in(1, 1)

        def step(t, _):
            for kk in range(_NSLOT):
                c = _NSLOT * t + kk
                wait_in(kk, c)
                compute(kk)
                start_out(kk, c)
                ns = (kk + 2) % _NSLOT

                @pl.when(c >= 2)
                def _():
                    wait_out(ns, c - 2)

                @pl.when(c + 2 < n_chunks)
                def _():
                    start_in(ns, c + 2)

            return 0

        lax.fori_loop(0, n_chunks // _NSLOT, step, 0)
        wait_out((n_chunks - 2) % _NSLOT, n_chunks - 2)
        wait_out((n_chunks - 1) % _NSLOT, n_chunks - 1)

    return k(x2d, emb).reshape(B, S, D)


def _copy_probe(x, emb_weight):
    B, S, D = x.shape
    grid = (S // _BS, B)
    return pl.pallas_call(
        lambda x_ref, o_ref: o_ref.__setitem__(Ellipsis, x_ref[...]),
        grid=grid,
        in_specs=[pl.BlockSpec((1, _BS, D), lambda i, j: (j, i, 0))],
        out_specs=pl.BlockSpec((1, _BS, D), lambda i, j: (j, i, 0)),
        out_shape=jax.ShapeDtypeStruct((B, S, D), x.dtype),
    )(x)


def kernel(x, emb_weight):
    return _kernel_sc_pipe(x, emb_weight)
